# baseline (speedup 1.0000x reference)
import jax
import jax.numpy as jnp
from jax.experimental import pallas as pl

_BM = 256


def _support_body(x_ref, w_ref, o_ref):
    o_ref[...] = jnp.dot(x_ref[...], w_ref[...], preferred_element_type=jnp.float32)


def _gc_mid_body(adj_ref, s_ref, b_ref, wn_ref, o_ref):
    h = jnp.dot(adj_ref[...], s_ref[...], preferred_element_type=jnp.float32)
    h = jnp.maximum(h + b_ref[...], 0.0)
    o_ref[...] = jnp.dot(h, wn_ref[...], preferred_element_type=jnp.float32)


def _gc_last_body(adj_ref, s_ref, b_ref, o_ref):
    h = jnp.dot(adj_ref[...], s_ref[...], preferred_element_type=jnp.float32)
    o_ref[...] = jnp.maximum(h + b_ref[...], 0.0)


def _support(x, w):
    n, d = x.shape
    c = w.shape[1]
    return pl.pallas_call(
        _support_body,
        grid=(n // _BM,),
        in_specs=[
            pl.BlockSpec((_BM, d), lambda i: (i, 0)),
            pl.BlockSpec((d, c), lambda i: (0, 0)),
        ],
        out_specs=pl.BlockSpec((_BM, c), lambda i: (i, 0)),
        out_shape=jax.ShapeDtypeStruct((n, c), jnp.float32),
    )(x, w)


def _gc_mid(adj, s, b, wn):
    n, c = s.shape
    cn = wn.shape[1]
    return pl.pallas_call(
        _gc_mid_body,
        grid=(n // _BM,),
        in_specs=[
            pl.BlockSpec((_BM, n), lambda i: (i, 0)),
            pl.BlockSpec((n, c), lambda i: (0, 0)),
            pl.BlockSpec((1, c), lambda i: (0, 0)),
            pl.BlockSpec((c, cn), lambda i: (0, 0)),
        ],
        out_specs=pl.BlockSpec((_BM, cn), lambda i: (i, 0)),
        out_shape=jax.ShapeDtypeStruct((n, cn), jnp.float32),
    )(adj, s, b.reshape(1, -1), wn)


def _gc_last(adj, s, b):
    n, c = s.shape
    return pl.pallas_call(
        _gc_last_body,
        grid=(n // _BM,),
        in_specs=[
            pl.BlockSpec((_BM, n), lambda i: (i, 0)),
            pl.BlockSpec((n, c), lambda i: (0, 0)),
            pl.BlockSpec((1, c), lambda i: (0, 0)),
        ],
        out_specs=pl.BlockSpec((_BM, c), lambda i: (i, 0)),
        out_shape=jax.ShapeDtypeStruct((n, c), jnp.float32),
    )(adj, s, b.reshape(1, -1))


def kernel(x, adj, W1, b1, W2, b2, W3, b3):
    s1 = _support(x, W1)
    s2 = _gc_mid(adj, s1, b1, W2)
    s3 = _gc_mid(adj, s2, b2, W3)
    return _gc_last(adj, s3, b3)
